# Initial kernel scaffold; baseline (speedup 1.0000x reference)
import jax
import jax.numpy as jnp
from jax.experimental import pallas as pl


def kernel(x_0, incidence_indices, W, b, eps):
    raise NotImplementedError("write your pallas kernel here")


# speedup vs baseline: 7.1775x; 7.1775x over previous
import functools

import jax
import jax.numpy as jnp
from jax import lax
from jax.experimental import pallas as pl
from jax.experimental.pallas import tpu as pltpu
from jax.experimental.pallas import tpu_sc as plsc

NC = 2
NS = 16
NW = NC * NS

CH = 80
ZR = 40
PAD_N = 10240


def _sc_segment_sum(table, gidx3, sidx3, n_ch):
  d = table.shape[1]
  n_out = PAD_N
  rows_pt = n_out // NS
  mesh = plsc.VectorSubcoreMesh(core_axis_name="c", subcore_axis_name="s")

  @functools.partial(
      pl.kernel,
      out_type=jax.ShapeDtypeStruct((NC, n_out, d), jnp.float32),
      mesh=mesh,
      scratch_types=[
          pltpu.VMEM((n_ch, CH), jnp.int32),
          pltpu.VMEM((n_ch, CH), jnp.int32),
          pltpu.VMEM((CH, d), jnp.float32),
          pltpu.VMEM((ZR, d), jnp.float32),
          pltpu.VMEM_SHARED((n_out, d), jnp.float32),
          pltpu.SemaphoreType.DMA,
      ],
  )
  def k(tbl_hbm, gidx_hbm, sidx_hbm, out_hbm, gv, sv, rows, zbuf, acc, sem):
    c = lax.axis_index("c")
    s = lax.axis_index("s")
    wid = c * NS + s

    pltpu.sync_copy(gidx_hbm.at[wid], gv)
    pltpu.sync_copy(sidx_hbm.at[wid], sv)

    zero16 = jnp.zeros((16,), jnp.float32)

    def zrow(i, _):
      def zcol(j, _):
        zbuf[i, pl.ds(j * 16, 16)] = zero16
        return 0
      return lax.fori_loop(0, d // 16, zcol, 0)

    lax.fori_loop(0, ZR, zrow, 0)

    def zcopy(r, _):
      pltpu.sync_copy(zbuf, acc.at[pl.ds(s * rows_pt + r * ZR, ZR)])
      return 0

    lax.fori_loop(0, rows_pt // ZR, zcopy, 0)
    plsc.subcore_barrier()

    def body(j, _):
      pltpu.async_copy(tbl_hbm.at[gv.at[j]], rows, sem).wait()
      pltpu.sync_copy(rows, acc.at[sv.at[j]], add=True)
      return 0

    lax.fori_loop(0, n_ch, body, 0)
    plsc.subcore_barrier()

    def wcopy(r, _):
      rs = s * rows_pt + r * ZR
      pltpu.sync_copy(acc.at[pl.ds(rs, ZR)], zbuf)
      pltpu.sync_copy(zbuf, out_hbm.at[c, pl.ds(rs, ZR)])
      return 0

    lax.fori_loop(0, rows_pt // ZR, wcopy, 0)

  return k(table, gidx3, sidx3)


def _combine(p, n):
  d = p.shape[2]
  bm = 1000

  def body(p_ref, o_ref):
    o_ref[...] = p_ref[0] + p_ref[1]

  return pl.pallas_call(
      body,
      grid=(n // bm,),
      in_specs=[pl.BlockSpec((NC, bm, d), lambda i: (0, i, 0))],
      out_specs=pl.BlockSpec((bm, d), lambda i: (i, 0)),
      out_shape=jax.ShapeDtypeStruct((n, d), jnp.float32),
  )(p)


def _gin_update(x0, q, w, b2, eps2):
  n, d = x0.shape
  bm = 1000

  def body(eps_ref, x_ref, q_ref, w_ref, b_ref, o_ref):
    scale = 1.0 + eps_ref[0, 0]
    a = x_ref[...] * scale + q_ref[0] + q_ref[1]
    o_ref[...] = lax.dot_general(
        a, w_ref[...], (((1,), (1,)), ((), ())),
        preferred_element_type=jnp.float32) + b_ref[...]

  return pl.pallas_call(
      body,
      grid=(n // bm,),
      in_specs=[
          pl.BlockSpec(memory_space=pltpu.SMEM),
          pl.BlockSpec((bm, d), lambda i: (i, 0)),
          pl.BlockSpec((NC, bm, d), lambda i: (0, i, 0)),
          pl.BlockSpec((d, d), lambda i: (0, 0)),
          pl.BlockSpec((1, d), lambda i: (0, 0)),
      ],
      out_specs=pl.BlockSpec((bm, d), lambda i: (i, 0)),
      out_shape=jax.ShapeDtypeStruct((n, d), jnp.float32),
  )(eps2, x0, q, w, b2)


def kernel(x_0, incidence_indices, W, b, eps):
  n_nodes, d = x_0.shape
  nnz = incidence_indices.shape[1]
  n_hedges = n_nodes
  per_w = nnz // NW
  n_ch = per_w // CH

  node_idx = incidence_indices[0]
  edge_idx = incidence_indices[1]
  nidx3 = node_idx.reshape(NW, n_ch, CH)
  eidx3 = edge_idx.reshape(NW, n_ch, CH)

  p = _sc_segment_sum(x_0, nidx3, eidx3, n_ch)
  x_1 = _combine(p, n_hedges)
  q = _sc_segment_sum(x_1, eidx3, nidx3, n_ch)
  x0_out = _gin_update(x_0, q, W, b.reshape(1, d), eps.reshape(1, 1))
  return (x0_out, x_1)
